# Initial kernel scaffold; baseline (speedup 1.0000x reference)
import jax
import jax.numpy as jnp
from jax.experimental import pallas as pl


def kernel(features, edge_index, norm, W, b, gamma, beta):
    raise NotImplementedError("write your pallas kernel here")


# speedup vs baseline: 6.5048x; 6.5048x over previous
import functools

import jax
import jax.numpy as jnp
from jax import lax
from jax.experimental import pallas as pl
from jax.experimental.pallas import tpu as pltpu
from jax.experimental.pallas import tpu_sc as plsc

N = 10000
E = 320000
D = 128

NC = 2
NS = 16
NW = NC * NS

EDGES_PER_W = E // NW
CH = 128
FULL_CHUNKS = EDGES_PER_W // CH
TAIL = EDGES_PER_W - FULL_CHUNKS * CH

ROWS_PER_TILE = 624
ZCH = 104
ZREPS = ROWS_PER_TILE // ZCH
ROWS_TAIL = N - NS * ROWS_PER_TILE


def _scale_body(f_ref, n_ref, o_ref):
    o_ref[...] = f_ref[...] * n_ref[...]


def _scale(features, norm):
    return pl.pallas_call(
        _scale_body,
        out_shape=jax.ShapeDtypeStruct((N, D), jnp.float32),
    )(features, norm)


def _sc_seg_sum_body(x_hbm, src_hbm, dst_hbm, out_hbm,
                     idx_s, idx_d, idx_st, idx_dt, rows, acc, sem):
    c = lax.axis_index("c")
    s = lax.axis_index("s")
    w = c * NS + s

    z16 = jnp.zeros((16,), jnp.float32)

    def zrows(i, carry):
        r = i // (D // 16)
        col = (i % (D // 16)) * 16
        rows[r, pl.ds(col, 16)] = z16
        return carry

    lax.fori_loop(0, CH * (D // 16), zrows, 0)

    def zacc(k, carry):
        base = s * ROWS_PER_TILE + k * ZCH
        pltpu.sync_copy(rows.at[pl.ds(0, ZCH)], acc.at[pl.ds(base, ZCH)])
        return carry

    lax.fori_loop(0, ZREPS, zacc, 0)

    @pl.when(s == 0)
    def _():
        pltpu.sync_copy(rows.at[pl.ds(0, ROWS_TAIL)],
                        acc.at[pl.ds(NS * ROWS_PER_TILE, ROWS_TAIL)])

    plsc.subcore_barrier()

    e0 = w * EDGES_PER_W

    def body(i, carry):
        off = e0 + i * CH
        pltpu.sync_copy(src_hbm.at[pl.ds(off, CH)], idx_s)
        pltpu.sync_copy(dst_hbm.at[pl.ds(off, CH)], idx_d)
        pltpu.async_copy(x_hbm.at[idx_s], rows, sem).wait()
        pltpu.sync_copy(rows, acc.at[idx_d], add=True)
        return carry

    lax.fori_loop(0, FULL_CHUNKS, body, 0)

    offt = e0 + FULL_CHUNKS * CH
    pltpu.sync_copy(src_hbm.at[pl.ds(offt, TAIL)], idx_st)
    pltpu.sync_copy(dst_hbm.at[pl.ds(offt, TAIL)], idx_dt)
    pltpu.async_copy(x_hbm.at[idx_st], rows.at[pl.ds(0, TAIL)], sem).wait()
    pltpu.sync_copy(rows.at[pl.ds(0, TAIL)], acc.at[idx_dt], add=True)

    plsc.subcore_barrier()

    def wb(k, carry):
        base = s * ROWS_PER_TILE + k * ZCH
        pltpu.sync_copy(acc.at[pl.ds(base, ZCH)],
                        out_hbm.at[c, pl.ds(base, ZCH)])
        return carry

    lax.fori_loop(0, ZREPS, wb, 0)

    @pl.when(s == 0)
    def _():
        pltpu.sync_copy(acc.at[pl.ds(NS * ROWS_PER_TILE, ROWS_TAIL)],
                        out_hbm.at[c, pl.ds(NS * ROWS_PER_TILE, ROWS_TAIL)])


def _sc_seg_sum(x, src, dst):
    mesh = plsc.VectorSubcoreMesh(core_axis_name="c", subcore_axis_name="s")
    return pl.kernel(
        _sc_seg_sum_body,
        mesh=mesh,
        out_type=jax.ShapeDtypeStruct((NC, N, D), jnp.float32),
        scratch_types=[
            pltpu.VMEM((CH,), jnp.int32),
            pltpu.VMEM((CH,), jnp.int32),
            pltpu.VMEM((TAIL,), jnp.int32),
            pltpu.VMEM((TAIL,), jnp.int32),
            pltpu.VMEM((CH, D), jnp.float32),
            pltpu.VMEM_SHARED((N, D), jnp.float32),
            pltpu.SemaphoreType.DMA,
        ],
    )(x, src, dst)


def _dense_body(p_ref, f_ref, n_ref, w_ref, b_ref, g_ref, be_ref, o_ref):
    agg = p_ref[0] + p_ref[1]
    h = jnp.dot(agg, w_ref[...], preferred_element_type=jnp.float32)
    h = h + b_ref[...]
    h = h * n_ref[...]
    mean = jnp.mean(h, axis=0, keepdims=True)
    var = jnp.mean((h - mean) ** 2, axis=0, keepdims=True)
    h = (h - mean) / jnp.sqrt(var + 1e-5) * g_ref[...] + be_ref[...]
    h = jnp.maximum(h, 0.0)
    o_ref[...] = f_ref[...] + h


def _dense(partials, features, norm, W, b, gamma, beta):
    return pl.pallas_call(
        _dense_body,
        out_shape=jax.ShapeDtypeStruct((N, D), jnp.float32),
    )(partials, features, norm, W,
      b.reshape(1, D), gamma.reshape(1, D), beta.reshape(1, D))


def kernel(features, edge_index, norm, W, b, gamma, beta):
    edge_index = edge_index.astype(jnp.int32)
    src = edge_index[0]
    dst = edge_index[1]
    x = _scale(features, norm)
    partials = _sc_seg_sum(x, src, dst)
    return _dense(partials, features, norm, W, b, gamma, beta)
